# Initial kernel scaffold; baseline (speedup 1.0000x reference)
import functools

import jax
import jax.numpy as jnp
from jax import lax
from jax.experimental import pallas as pl
from jax.experimental.pallas import tpu as pltpu
from jax.experimental.pallas import tpu_sc as plsc

N_DEV = 10000
DEG = 16
N_BRE = 80000
EMB = 128

NW = 32
PER_W = 320
N_PAD = NW * PER_W
CH = 64
N_CH = PER_W // CH

TC_BLK = 256


def _sc_body(tbl_hbm, devt_hbm, vpre_hbm, ne_hbm, cbs_hbm,
             dev_t, tg, nb_v, cbs_v, acc, sem):
    wid = lax.axis_index("s") * 2 + lax.axis_index("c")
    base = wid * PER_W

    pltpu.sync_copy(devt_hbm.at[:, pl.ds(base, PER_W)], dev_t)

    iota16 = lax.iota(jnp.int32, 16)
    col0 = jnp.zeros((16,), jnp.int32)
    col1 = jnp.full((16,), 1, jnp.int32)
    col2 = jnp.full((16,), 2, jnp.int32)

    def d_body(d, carry):
        cps = [
            pltpu.async_copy(
                tbl_hbm.at[dev_t.at[d, pl.ds(ci * CH, CH)]], tg.at[ci], sem)
            for ci in range(N_CH)
        ]
        for ci in range(N_CH):
            cps[ci].wait()
            for g in range(CH // 16):
                rows = g * 16 + iota16
                b0 = plsc.load_gather(tg.at[ci], [rows, col0])
                b1 = plsc.load_gather(tg.at[ci], [rows, col1])
                bs = plsc.load_gather(tg.at[ci], [rows, col2])
                nloc = ci * CH + g * 16 + iota16
                nid = base + nloc
                nb16 = jnp.where(b0 != nid, b0, b1)
                nb_v[d, pl.ds(ci * CH + g * 16, 16)] = nb16
                plsc.store_scatter(
                    cbs_v, [nloc, jnp.full((16,), d, jnp.int32)],
                    plsc.bitcast(bs, jnp.float32))
        return carry

    lax.fori_loop(0, DEG, d_body, 0, unroll=False)

    cps = [
        pltpu.async_copy(
            vpre_hbm.at[nb_v.at[0, pl.ds(ci * CH, CH)]],
            acc.at[pl.ds(ci * CH, CH)], sem)
        for ci in range(N_CH)
    ]
    for cp in cps:
        cp.wait()

    def d_acc(d, carry):
        cps = [
            pltpu.async_copy(
                vpre_hbm.at[nb_v.at[d, pl.ds(ci * CH, CH)]],
                acc.at[pl.ds(ci * CH, CH)], sem, add=True)
            for ci in range(N_CH)
        ]
        for cp in cps:
            cp.wait()
        return carry

    lax.fori_loop(1, DEG, d_acc, 0, unroll=False)

    pltpu.sync_copy(acc, ne_hbm.at[pl.ds(base, PER_W)])
    pltpu.sync_copy(cbs_v, cbs_hbm.at[pl.ds(base, PER_W)])


@jax.jit
def _sc_gather(tbl, devt, vpre):
    mesh = plsc.VectorSubcoreMesh(core_axis_name="c", subcore_axis_name="s")
    fn = functools.partial(
        pl.kernel,
        out_type=(
            jax.ShapeDtypeStruct((N_PAD, EMB), jnp.float32),
            jax.ShapeDtypeStruct((N_PAD, DEG), jnp.float32),
        ),
        mesh=mesh,
        scratch_types=[
            pltpu.VMEM((DEG, PER_W), jnp.int32),
            pltpu.VMEM((N_CH, CH, 4), jnp.int32),
            pltpu.VMEM((DEG, PER_W), jnp.int32),
            pltpu.VMEM((PER_W, DEG), jnp.float32),
            pltpu.VMEM((PER_W, EMB), jnp.float32),
            pltpu.SemaphoreType.DMA,
        ],
    )(_sc_body)
    return fn(tbl, devt, vpre)


def _tc_body(ne_ref, cbs_ref, ps_ref, w0t, w1r, w2r, w3t, w4r, w5t,
             bias, wcb, out_ref):
    cbs = cbs_ref[...]
    ps = ps_ref[...]
    ne = ne_ref[...]

    b0r = bias[0:1, :]
    b1r = bias[1:2, :]
    b2r = bias[2:3, :]
    b3r = bias[3:4, :]
    b4r = bias[4:5, :]
    b5r = bias[5:6, :]

    w4 = w4r[...]
    be = jnp.tanh(cbs[:, 0:1] * w4 + b4r)
    for d in range(1, DEG):
        be = be + jnp.tanh(cbs[:, d:d + 1] * w4 + b4r)
    breaker = jnp.tanh(
        jnp.dot(be, w3t[...], preferred_element_type=jnp.float32) + b3r)

    tmp = jnp.sum(cbs, axis=1, keepdims=True)
    w1 = w1r[...]
    pe = jnp.tanh(ps[:, 0:1] * w1 + b1r)
    for i in range(1, 3):
        pe = pe + jnp.tanh(ps[:, i:i + 1] * w1 + b1r)
    pe = pe + 3.0 * jnp.tanh(tmp * w2r[...] + b2r)
    protector = jnp.tanh(
        jnp.dot(pe, w0t[...], preferred_element_type=jnp.float32) + b0r)

    neighbor = jnp.tanh(
        jnp.dot(ne, w5t[...], preferred_element_type=jnp.float32) + b5r)

    wc = wcb[...]
    out_ref[...] = jnp.tanh(
        protector * wc[0:1, :] + breaker * wc[1:2, :]
        + neighbor * wc[2:3, :] + wc[3:4, :])


@jax.jit
def _tc_dense(ne, cbs, ps, w0t, w1r, w2r, w3t, w4r, w5t, bias, wcb):
    grid = (N_PAD // TC_BLK,)
    full = lambda shape: pl.BlockSpec(shape, lambda i: (0, 0))
    return pl.pallas_call(
        _tc_body,
        grid=grid,
        in_specs=[
            pl.BlockSpec((TC_BLK, EMB), lambda i: (i, 0)),
            pl.BlockSpec((TC_BLK, DEG), lambda i: (i, 0)),
            pl.BlockSpec((TC_BLK, 4), lambda i: (i, 0)),
            full((EMB, EMB)), full((1, EMB)), full((1, EMB)),
            full((EMB, EMB)), full((1, EMB)), full((EMB, EMB)),
            full((8, EMB)), full((8, EMB)),
        ],
        out_specs=pl.BlockSpec((TC_BLK, EMB), lambda i: (i, 0)),
        out_shape=jax.ShapeDtypeStruct((N_PAD, EMB), jnp.float32),
    )(ne, cbs, ps, w0t, w1r, w2r, w3t, w4r, w5t, bias, wcb)


def kernel(V_pre, devices, breakers, protector_sate, breaker_state,
           W0, b0, W1, b1, W2, b2, W3, b3, W4, b4, W5, b5, Wc, bc):
    dev = devices.astype(jnp.int32)
    br = breakers.astype(jnp.int32)
    bs_bits = lax.bitcast_convert_type(breaker_state, jnp.int32)
    tbl = jnp.concatenate(
        [br, bs_bits[:, None], jnp.zeros((N_BRE, 1), jnp.int32)], axis=1)

    dev_p = jnp.pad(dev, ((0, N_PAD - N_DEV), (0, 0)))
    devt = dev_p.T
    ps_p = jnp.pad(protector_sate, ((0, N_PAD - N_DEV), (0, 1)))

    ne, cbs = _sc_gather(tbl, devt, V_pre)

    row = lambda v: v.reshape(1, EMB)
    bias = jnp.concatenate(
        [row(b0), row(b1), row(b2), row(b3), row(b4), row(b5),
         jnp.zeros((2, EMB), jnp.float32)], axis=0)
    wcb = jnp.concatenate(
        [jnp.broadcast_to(Wc[0], (1, EMB)), jnp.broadcast_to(Wc[1], (1, EMB)),
         jnp.broadcast_to(Wc[2], (1, EMB)), jnp.broadcast_to(bc[0], (1, EMB)),
         jnp.zeros((4, EMB), jnp.float32)], axis=0)

    out = _tc_dense(ne, cbs, ps_p, W0.T, W1.T, W2.T, W3.T, W4.T, W5.T,
                    bias, wcb)
    return out[:N_DEV]


# speedup vs baseline: 5.0506x; 5.0506x over previous
import functools

import jax
import jax.numpy as jnp
from jax import lax
from jax.experimental import pallas as pl
from jax.experimental.pallas import tpu as pltpu
from jax.experimental.pallas import tpu_sc as plsc

N_DEV = 10000
DEG = 16
N_BRE = 80000
EMB = 128

NW = 32
PER_W = 320
N_PAD = NW * PER_W
CH = 64
N_CH = PER_W // CH

TC_BLK = 256


def _sc_body(b0_hbm, b1_hbm, bs_hbm, devt_hbm, vpre_hbm, ne_hbm, cbs_hbm,
             dev_t, tb0, tb1, tbs, nb_v, cbs_v, acc, sem):
    wid = lax.axis_index("s") * 2 + lax.axis_index("c")
    base = wid * PER_W

    pltpu.sync_copy(devt_hbm.at[wid], dev_t)

    iota16 = lax.iota(jnp.int32, 16)

    def d_body(d, carry):
        cps = []
        for ci in range(N_CH):
            idx = dev_t.at[d, pl.ds(ci * CH, CH)]
            cps.append(pltpu.async_copy(b0_hbm.at[idx], tb0.at[ci], sem))
            cps.append(pltpu.async_copy(b1_hbm.at[idx], tb1.at[ci], sem))
            cps.append(pltpu.async_copy(bs_hbm.at[idx], tbs.at[ci], sem))
        for cp in cps:
            cp.wait()
        for ci in range(N_CH):
            for g in range(CH // 16):
                sl = pl.ds(g * 16, 16)
                e0 = tb0[ci, sl]
                e1 = tb1[ci, sl]
                nid = base + ci * CH + g * 16 + iota16
                dst = pl.ds(ci * CH + g * 16, 16)
                nb_v[d, dst] = jnp.where(e0 != nid, e0, e1)
                cbs_v[d, dst] = tbs[ci, sl]
        return carry

    lax.fori_loop(0, DEG, d_body, 0, unroll=False)

    cps = [
        pltpu.async_copy(
            vpre_hbm.at[nb_v.at[0, pl.ds(ci * CH, CH)]],
            acc.at[pl.ds(ci * CH, CH)], sem)
        for ci in range(N_CH)
    ]
    for cp in cps:
        cp.wait()

    def d_acc(d, carry):
        cps = [
            pltpu.async_copy(
                vpre_hbm.at[nb_v.at[d, pl.ds(ci * CH, CH)]],
                acc.at[pl.ds(ci * CH, CH)], sem, add=True)
            for ci in range(N_CH)
        ]
        for cp in cps:
            cp.wait()
        return carry

    lax.fori_loop(1, DEG, d_acc, 0, unroll=False)

    pltpu.sync_copy(acc, ne_hbm.at[pl.ds(base, PER_W)])
    pltpu.sync_copy(cbs_v, cbs_hbm.at[wid])


@jax.jit
def _sc_gather(b0t, b1t, bst, devt, vpre):
    mesh = plsc.VectorSubcoreMesh(core_axis_name="c", subcore_axis_name="s")
    fn = functools.partial(
        pl.kernel,
        out_type=(
            jax.ShapeDtypeStruct((N_PAD, EMB), jnp.float32),
            jax.ShapeDtypeStruct((NW, DEG, PER_W), jnp.float32),
        ),
        mesh=mesh,
        scratch_types=[
            pltpu.VMEM((DEG, PER_W), jnp.int32),
            pltpu.VMEM((N_CH, CH), jnp.int32),
            pltpu.VMEM((N_CH, CH), jnp.int32),
            pltpu.VMEM((N_CH, CH), jnp.float32),
            pltpu.VMEM((DEG, PER_W), jnp.int32),
            pltpu.VMEM((DEG, PER_W), jnp.float32),
            pltpu.VMEM((PER_W, EMB), jnp.float32),
            pltpu.SemaphoreType.DMA,
        ],
    )(_sc_body)
    return fn(b0t, b1t, bst, devt, vpre)


def _tc_body(ne_ref, cbs_ref, ps_ref, w0t, w1r, w2r, w3t, w4r, w5t,
             bias, wcb, out_ref):
    cbs = cbs_ref[...]
    ps = ps_ref[...]
    ne = ne_ref[...]

    b0r = bias[0:1, :]
    b1r = bias[1:2, :]
    b2r = bias[2:3, :]
    b3r = bias[3:4, :]
    b4r = bias[4:5, :]
    b5r = bias[5:6, :]

    w4 = w4r[...]
    be = jnp.tanh(cbs[:, 0:1] * w4 + b4r)
    for d in range(1, DEG):
        be = be + jnp.tanh(cbs[:, d:d + 1] * w4 + b4r)
    breaker = jnp.tanh(
        jnp.dot(be, w3t[...], preferred_element_type=jnp.float32) + b3r)

    tmp = jnp.sum(cbs, axis=1, keepdims=True)
    w1 = w1r[...]
    pe = jnp.tanh(ps[:, 0:1] * w1 + b1r)
    for i in range(1, 3):
        pe = pe + jnp.tanh(ps[:, i:i + 1] * w1 + b1r)
    pe = pe + 3.0 * jnp.tanh(tmp * w2r[...] + b2r)
    protector = jnp.tanh(
        jnp.dot(pe, w0t[...], preferred_element_type=jnp.float32) + b0r)

    neighbor = jnp.tanh(
        jnp.dot(ne, w5t[...], preferred_element_type=jnp.float32) + b5r)

    wc = wcb[...]
    out_ref[...] = jnp.tanh(
        protector * wc[0:1, :] + breaker * wc[1:2, :]
        + neighbor * wc[2:3, :] + wc[3:4, :])


@jax.jit
def _tc_dense(ne, cbs, ps, w0t, w1r, w2r, w3t, w4r, w5t, bias, wcb):
    grid = (N_PAD // TC_BLK,)
    full = lambda shape: pl.BlockSpec(shape, lambda i: (0, 0))
    return pl.pallas_call(
        _tc_body,
        grid=grid,
        in_specs=[
            pl.BlockSpec((TC_BLK, EMB), lambda i: (i, 0)),
            pl.BlockSpec((TC_BLK, DEG), lambda i: (i, 0)),
            pl.BlockSpec((TC_BLK, 4), lambda i: (i, 0)),
            full((EMB, EMB)), full((1, EMB)), full((1, EMB)),
            full((EMB, EMB)), full((1, EMB)), full((EMB, EMB)),
            full((8, EMB)), full((8, EMB)),
        ],
        out_specs=pl.BlockSpec((TC_BLK, EMB), lambda i: (i, 0)),
        out_shape=jax.ShapeDtypeStruct((N_PAD, EMB), jnp.float32),
    )(ne, cbs, ps, w0t, w1r, w2r, w3t, w4r, w5t, bias, wcb)


def kernel(V_pre, devices, breakers, protector_sate, breaker_state,
           W0, b0, W1, b1, W2, b2, W3, b3, W4, b4, W5, b5, Wc, bc):
    dev = devices.astype(jnp.int32)
    br = breakers.astype(jnp.int32)
    b0t = br[:, 0]
    b1t = br[:, 1]

    dev_p = jnp.pad(dev, ((0, N_PAD - N_DEV), (0, 0)))
    devt = dev_p.T.reshape(DEG, NW, PER_W).transpose(1, 0, 2)
    ps_p = jnp.pad(protector_sate, ((0, N_PAD - N_DEV), (0, 1)))

    ne, cbs3 = _sc_gather(b0t, b1t, breaker_state, devt, V_pre)
    cbs = cbs3.transpose(0, 2, 1).reshape(N_PAD, DEG)

    row = lambda v: v.reshape(1, EMB)
    bias = jnp.concatenate(
        [row(b0), row(b1), row(b2), row(b3), row(b4), row(b5),
         jnp.zeros((2, EMB), jnp.float32)], axis=0)
    wcb = jnp.concatenate(
        [jnp.broadcast_to(Wc[0], (1, EMB)), jnp.broadcast_to(Wc[1], (1, EMB)),
         jnp.broadcast_to(Wc[2], (1, EMB)), jnp.broadcast_to(bc[0], (1, EMB)),
         jnp.zeros((4, EMB), jnp.float32)], axis=0)

    out = _tc_dense(ne, cbs, ps_p, W0.T, W1.T, W2.T, W3.T, W4.T, W5.T,
                    bias, wcb)
    return out[:N_DEV]


# speedup vs baseline: 5.4848x; 1.0860x over previous
import functools

import jax
import jax.numpy as jnp
from jax import lax
from jax.experimental import pallas as pl
from jax.experimental.pallas import tpu as pltpu
from jax.experimental.pallas import tpu_sc as plsc

N_DEV = 10000
DEG = 16
N_BRE = 80000
EMB = 128

NW = 32
PER_W = 320
N_PAD = NW * PER_W
CH = 80
N_CH = PER_W // CH
NJ = DEG * N_CH

TC_BLK = 256


def _sc_body(b0_hbm, b1_hbm, bs_hbm, devt_hbm, vpre_hbm, ne_hbm, cbs_hbm,
             dev_t, tb0, tb1, tbs, nb_v, cbs_v, acc, sem_t, sem_v):
    wid = lax.axis_index("s") * 2 + lax.axis_index("c")
    base = wid * PER_W

    pltpu.sync_copy(devt_hbm.at[wid], dev_t)

    iota16 = lax.iota(jnp.int32, 16)

    def fire_tables(d):
        for ci in range(N_CH):
            idx = dev_t.at[d * N_CH + ci]
            pltpu.async_copy(b0_hbm.at[idx], tb0.at[d * N_CH + ci], sem_t)
            pltpu.async_copy(b1_hbm.at[idx], tb1.at[d * N_CH + ci], sem_t)
            pltpu.async_copy(bs_hbm.at[idx], tbs.at[d * N_CH + ci], sem_t)

    def drain_tables():
        for _ in range(N_CH):
            pltpu.make_async_copy(
                b0_hbm.at[pl.ds(0, CH)], tb0.at[0], sem_t).wait()
            pltpu.make_async_copy(
                b1_hbm.at[pl.ds(0, CH)], tb1.at[0], sem_t).wait()
            pltpu.make_async_copy(
                bs_hbm.at[pl.ds(0, CH)], tbs.at[0], sem_t).wait()

    def compute(d):
        for ci in range(N_CH):
            j = d * N_CH + ci
            for g in range(CH // 16):
                sl = pl.ds(g * 16, 16)
                e0 = tb0[j, sl]
                e1 = tb1[j, sl]
                nid = base + ci * CH + g * 16 + iota16
                nb_v[j, sl] = jnp.where(e0 != nid, e0, e1)
                cbs_v[d, pl.ds(ci * CH + g * 16, 16)] = tbs[j, sl]

    def fire_vpre(d, add):
        for ci in range(N_CH):
            pltpu.async_copy(
                vpre_hbm.at[nb_v.at[d * N_CH + ci]],
                acc.at[pl.ds(ci * CH, CH)], sem_v, add=add)

    def drain_vpre():
        for ci in range(N_CH):
            pltpu.make_async_copy(
                vpre_hbm.at[pl.ds(0, CH)],
                acc.at[pl.ds(ci * CH, CH)], sem_v).wait()

    fire_tables(0)
    drain_tables()
    compute(0)
    fire_vpre(0, add=False)

    def d_body(d, carry):
        fire_tables(d)
        drain_tables()
        compute(d)
        drain_vpre()
        fire_vpre(d, add=True)
        return carry

    lax.fori_loop(1, DEG, d_body, 0, unroll=False)
    drain_vpre()

    pltpu.sync_copy(acc, ne_hbm.at[pl.ds(base, PER_W)])
    pltpu.sync_copy(cbs_v, cbs_hbm.at[wid])


@jax.jit
def _sc_gather(b0t, b1t, bst, devt, vpre):
    mesh = plsc.VectorSubcoreMesh(core_axis_name="c", subcore_axis_name="s")
    fn = functools.partial(
        pl.kernel,
        out_type=(
            jax.ShapeDtypeStruct((N_PAD, EMB), jnp.float32),
            jax.ShapeDtypeStruct((NW, DEG, PER_W), jnp.float32),
        ),
        mesh=mesh,
        scratch_types=[
            pltpu.VMEM((NJ, CH), jnp.int32),
            pltpu.VMEM((NJ, CH), jnp.int32),
            pltpu.VMEM((NJ, CH), jnp.int32),
            pltpu.VMEM((NJ, CH), jnp.float32),
            pltpu.VMEM((NJ, CH), jnp.int32),
            pltpu.VMEM((DEG, PER_W), jnp.float32),
            pltpu.VMEM((PER_W, EMB), jnp.float32),
            pltpu.SemaphoreType.DMA,
            pltpu.SemaphoreType.DMA,
        ],
    )(_sc_body)
    return fn(b0t, b1t, bst, devt, vpre)


def _tc_body(ne_ref, cbs_ref, ps_ref, w0t, w1r, w2r, w3t, w4r, w5t,
             bias, wcb, out_ref):
    cbs = cbs_ref[...]
    ps = ps_ref[...]
    ne = ne_ref[...]

    b0r = bias[0:1, :]
    b1r = bias[1:2, :]
    b2r = bias[2:3, :]
    b3r = bias[3:4, :]
    b4r = bias[4:5, :]
    b5r = bias[5:6, :]

    w4 = w4r[...]
    be = jnp.tanh(cbs[:, 0:1] * w4 + b4r)
    for d in range(1, DEG):
        be = be + jnp.tanh(cbs[:, d:d + 1] * w4 + b4r)
    breaker = jnp.tanh(
        jnp.dot(be, w3t[...], preferred_element_type=jnp.float32) + b3r)

    tmp = jnp.sum(cbs, axis=1, keepdims=True)
    w1 = w1r[...]
    pe = jnp.tanh(ps[:, 0:1] * w1 + b1r)
    for i in range(1, 3):
        pe = pe + jnp.tanh(ps[:, i:i + 1] * w1 + b1r)
    pe = pe + 3.0 * jnp.tanh(tmp * w2r[...] + b2r)
    protector = jnp.tanh(
        jnp.dot(pe, w0t[...], preferred_element_type=jnp.float32) + b0r)

    neighbor = jnp.tanh(
        jnp.dot(ne, w5t[...], preferred_element_type=jnp.float32) + b5r)

    wc = wcb[...]
    out_ref[...] = jnp.tanh(
        protector * wc[0:1, :] + breaker * wc[1:2, :]
        + neighbor * wc[2:3, :] + wc[3:4, :])


@jax.jit
def _tc_dense(ne, cbs, ps, w0t, w1r, w2r, w3t, w4r, w5t, bias, wcb):
    grid = (N_PAD // TC_BLK,)
    full = lambda shape: pl.BlockSpec(shape, lambda i: (0, 0))
    return pl.pallas_call(
        _tc_body,
        grid=grid,
        in_specs=[
            pl.BlockSpec((TC_BLK, EMB), lambda i: (i, 0)),
            pl.BlockSpec((TC_BLK, DEG), lambda i: (i, 0)),
            pl.BlockSpec((TC_BLK, 4), lambda i: (i, 0)),
            full((EMB, EMB)), full((1, EMB)), full((1, EMB)),
            full((EMB, EMB)), full((1, EMB)), full((EMB, EMB)),
            full((8, EMB)), full((8, EMB)),
        ],
        out_specs=pl.BlockSpec((TC_BLK, EMB), lambda i: (i, 0)),
        out_shape=jax.ShapeDtypeStruct((N_PAD, EMB), jnp.float32),
    )(ne, cbs, ps, w0t, w1r, w2r, w3t, w4r, w5t, bias, wcb)


def kernel(V_pre, devices, breakers, protector_sate, breaker_state,
           W0, b0, W1, b1, W2, b2, W3, b3, W4, b4, W5, b5, Wc, bc):
    dev = devices.astype(jnp.int32)
    br = breakers.astype(jnp.int32)
    b0t = br[:, 0]
    b1t = br[:, 1]

    dev_p = jnp.pad(dev, ((0, N_PAD - N_DEV), (0, 0)))
    devt = (dev_p.T.reshape(DEG, NW, N_CH, CH)
            .transpose(1, 0, 2, 3).reshape(NW, NJ, CH))
    ps_p = jnp.pad(protector_sate, ((0, N_PAD - N_DEV), (0, 1)))

    ne, cbs3 = _sc_gather(b0t, b1t, breaker_state, devt, V_pre)
    cbs = cbs3.transpose(0, 2, 1).reshape(N_PAD, DEG)

    row = lambda v: v.reshape(1, EMB)
    bias = jnp.concatenate(
        [row(b0), row(b1), row(b2), row(b3), row(b4), row(b5),
         jnp.zeros((2, EMB), jnp.float32)], axis=0)
    wcb = jnp.concatenate(
        [jnp.broadcast_to(Wc[0], (1, EMB)), jnp.broadcast_to(Wc[1], (1, EMB)),
         jnp.broadcast_to(Wc[2], (1, EMB)), jnp.broadcast_to(bc[0], (1, EMB)),
         jnp.zeros((4, EMB), jnp.float32)], axis=0)

    out = _tc_dense(ne, cbs, ps_p, W0.T, W1.T, W2.T, W3.T, W4.T, W5.T,
                    bias, wcb)
    return out[:N_DEV]
